# baseline (speedup 1.0000x reference)
import functools

import jax
import jax.numpy as jnp
from jax import lax
from jax.experimental import pallas as pl
from jax.experimental.pallas import tpu as pltpu
from jax.experimental.pallas import tpu_sc as plsc

N, D, E, TOP_K = 8192, 2048, 8, 2
H = D // 2
BLK = 256
M = TOP_K * N + E * BLK
NB = M // BLK
GB = 1024


def _gate_body(x_ref, gw_ref, gb_ref, itop_ref, wtop_ref, xp_ref):
    x = x_ref[...]
    logits = jax.lax.dot_general(
        x, gw_ref[...], (((1,), (1,)), ((), ())),
        preferred_element_type=jnp.float32) + gb_ref[...]
    probs = jax.nn.softmax(logits, axis=-1)
    cols = jax.lax.broadcasted_iota(jnp.int32, probs.shape, 1)
    i1 = jnp.argmax(probs, axis=-1, keepdims=True)
    p1 = jnp.max(probs, axis=-1, keepdims=True)
    pm = jnp.where(cols == i1, -jnp.inf, probs)
    i2 = jnp.argmax(pm, axis=-1, keepdims=True)
    p2 = jnp.max(pm, axis=-1, keepdims=True)
    itop_ref[...] = jnp.concatenate([i1, i2], axis=1)
    wtop_ref[...] = jnp.concatenate([p1, p2], axis=1)
    xbf = x.astype(jnp.bfloat16)
    lo = lax.bitcast_convert_type(xbf[:, :H], jnp.uint16).astype(jnp.uint32)
    hi = lax.bitcast_convert_type(xbf[:, H:], jnp.uint16).astype(jnp.uint32)
    xp_ref[...] = (hi << 16) | lo


def _gate(x, gate_w, gate_b):
    return pl.pallas_call(
        _gate_body,
        grid=(N // GB,),
        in_specs=[
            pl.BlockSpec((GB, D), lambda n: (n, 0)),
            pl.BlockSpec((E, D), lambda n: (0, 0)),
            pl.BlockSpec((1, E), lambda n: (0, 0)),
        ],
        out_specs=[
            pl.BlockSpec((GB, TOP_K), lambda n: (n, 0)),
            pl.BlockSpec((GB, TOP_K), lambda n: (n, 0)),
            pl.BlockSpec((GB, H), lambda n: (n, 0)),
        ],
        out_shape=[
            jax.ShapeDtypeStruct((N, TOP_K), jnp.int32),
            jax.ShapeDtypeStruct((N, TOP_K), jnp.float32),
            jax.ShapeDtypeStruct((N, H), jnp.uint32),
        ],
    )(x, gate_w, gate_b.reshape(1, E))


def _route(itop, wtop):
    flat_e = itop.reshape(-1)
    tok = jax.lax.iota(jnp.int32, TOP_K * N) // TOP_K
    oh = (flat_e[:, None] == jnp.arange(E, dtype=jnp.int32)[None, :])
    csum = jnp.cumsum(oh.astype(jnp.int32), axis=0)
    rank = jnp.take_along_axis(csum - 1, flat_e[:, None], axis=1)[:, 0]
    cnt = csum[-1]
    padded = ((cnt + BLK - 1) // BLK) * BLK
    off = jnp.concatenate([jnp.zeros((1,), jnp.int32),
                           jnp.cumsum(padded)[:-1].astype(jnp.int32)])
    pos = off[flat_e] + rank
    row_token = jnp.zeros((M,), jnp.int32).at[pos].set(tok)
    row_weight = jnp.zeros((M,), jnp.float32).at[pos].set(wtop.reshape(-1))
    block_expert = jnp.clip(
        jnp.searchsorted(off, jnp.arange(NB, dtype=jnp.int32) * BLK,
                         side="right") - 1, 0, E - 1).astype(jnp.int32)
    pos1 = pos[0::TOP_K]
    pos2 = pos[1::TOP_K]
    return row_token, row_weight, block_expert, pos1, pos2


NW = 32
CH = 32
G = 3


@functools.cache
def _make_gather(rows):
    rpw = rows // NW
    nchunk = rpw // CH

    def body(x_hbm, idx_hbm, out_hbm, idx_v, bufs, gsem, wsem):
        wid = lax.axis_index("s") * 2 + lax.axis_index("c")
        base = wid * rpw
        pltpu.sync_copy(idx_hbm.at[pl.ds(base, rpw)], idx_v)

        def group(g, _):
            cbase = g * G * CH
            cps = []
            for i in range(G):
                cb = cbase + i * CH
                cps.append(pltpu.async_copy(
                    x_hbm.at[idx_v.at[pl.ds(cb, CH)]], bufs[i], gsem))
            wps = []
            for i in range(G):
                cps[i].wait()
                wps.append(pltpu.async_copy(
                    bufs[i], out_hbm.at[pl.ds(base + cbase + i * CH, CH)],
                    wsem))
            for wp in wps:
                wp.wait()
            return 0

        lax.fori_loop(0, nchunk // G, group, 0)

    return pl.kernel(
        body,
        out_type=jax.ShapeDtypeStruct((rows, H), jnp.uint32),
        mesh=plsc.VectorSubcoreMesh(core_axis_name="c", subcore_axis_name="s"),
        scratch_types=[
            pltpu.VMEM((rpw,), jnp.int32),
            [pltpu.VMEM((CH, H), jnp.uint32)] * G,
            pltpu.SemaphoreType.DMA,
            pltpu.SemaphoreType.DMA,
        ],
    )


def _gather_rows(xp, row_token):
    return _make_gather(row_token.shape[0])(xp, row_token)


def _unpack_bf16(p):
    lo = lax.bitcast_convert_type(
        (p & jnp.uint32(0xFFFF)).astype(jnp.uint16), jnp.bfloat16)
    hi = lax.bitcast_convert_type(
        (p >> 16).astype(jnp.uint16), jnp.bfloat16)
    return lo, hi


def _mm_body(be_ref, xg_ref, w_ref, b_ref, rw_ref, yg_ref, wbf_s, laste_s):
    j = pl.program_id(0)
    be_j = be_ref[j]

    @pl.when(j == 0)
    def _init():
        laste_s[0] = jnp.int32(-1)

    @pl.when(be_j != laste_s[0])
    def _recast():
        wbf_s[...] = w_ref[0].astype(jnp.bfloat16)
        laste_s[0] = be_j

    xlo, xhi = _unpack_bf16(xg_ref[...])
    y = jax.lax.dot_general(xlo, wbf_s[:, :H], (((1,), (1,)), ((), ())),
                            preferred_element_type=jnp.float32)
    y = y + jax.lax.dot_general(xhi, wbf_s[:, H:], (((1,), (1,)), ((), ())),
                                preferred_element_type=jnp.float32)
    yg_ref[...] = (y + b_ref[0]) * rw_ref[...]


def _mm_body_alias(be_ref, xg_ref, w_ref, b_ref, rw_ref, ygin_ref, yg_ref,
                   wbf_s, laste_s):
    _mm_body(be_ref, xg_ref, w_ref, b_ref, rw_ref, yg_ref, wbf_s, laste_s)


def _grouped_mm_half(xgp_half, expert_w, expert_b, rw_half, be_half,
                     off, yg_in=None):
    nb = be_half.shape[0]
    in_specs = [
        pl.BlockSpec((BLK, H), lambda j, be: (j, 0)),
        pl.BlockSpec((1, D, D), lambda j, be: (be[j], 0, 0)),
        pl.BlockSpec((1, 1, D), lambda j, be: (be[j], 0, 0)),
        pl.BlockSpec((BLK, 1), lambda j, be: (j, 0)),
    ]
    args = [be_half, xgp_half, expert_w, expert_b.reshape(E, 1, D),
            rw_half.reshape(-1, 1)]
    aliases = {}
    if yg_in is not None:
        in_specs.append(pl.BlockSpec(memory_space=pl.ANY))
        args.append(yg_in)
        aliases = {5: 0}
    grid_spec = pltpu.PrefetchScalarGridSpec(
        num_scalar_prefetch=1,
        grid=(nb,),
        in_specs=in_specs,
        out_specs=pl.BlockSpec((BLK, D), lambda j, be: (off + j, 0)),
        scratch_shapes=[
            pltpu.VMEM((D, D), jnp.bfloat16),
            pltpu.SMEM((1,), jnp.int32),
        ],
    )
    return pl.pallas_call(
        _mm_body if yg_in is None else _mm_body_alias,
        grid_spec=grid_spec,
        out_shape=jax.ShapeDtypeStruct((M, D), jnp.float32),
        input_output_aliases=aliases,
    )(*args)


TPW = N // NW
CT = 8
NTC = TPW // CT


def _combine_body(yg_hbm, p1_hbm, p2_hbm, out_hbm,
                  p1v, p2v, bufa, bufb, obuf, gsem, wsem):
    wid = lax.axis_index("s") * 2 + lax.axis_index("c")
    tbase = wid * TPW
    pltpu.sync_copy(p1_hbm.at[pl.ds(tbase, TPW)], p1v)
    pltpu.sync_copy(p2_hbm.at[pl.ds(tbase, TPW)], p2v)

    def compute(a_ref, b_ref, o_ref):
        def kloop(k, _):
            kk = k * 16
            for r in range(CT):
                o_ref[r, pl.ds(kk, 16)] = (
                    a_ref[r, pl.ds(kk, 16)] + b_ref[r, pl.ds(kk, 16)])
            return 0
        lax.fori_loop(0, D // 16, kloop, 0)

    def pair(j, _):
        gps = []
        for s in range(2):
            c = (j * 2 + s) * CT
            gps.append((
                pltpu.async_copy(yg_hbm.at[p1v.at[pl.ds(c, CT)]], bufa[s], gsem),
                pltpu.async_copy(yg_hbm.at[p2v.at[pl.ds(c, CT)]], bufb[s], gsem)))
        wps = []
        for s in range(2):
            c = (j * 2 + s) * CT
            gps[s][0].wait()
            gps[s][1].wait()
            compute(bufa[s], bufb[s], obuf[s])
            wps.append(pltpu.async_copy(
                obuf[s], out_hbm.at[pl.ds(tbase + c, CT)], wsem))
        for wp in wps:
            wp.wait()
        return 0

    lax.fori_loop(0, NTC // 2, pair, 0)


@functools.cache
def _make_combine():
    return pl.kernel(
        _combine_body,
        out_type=jax.ShapeDtypeStruct((N, D), jnp.float32),
        mesh=plsc.VectorSubcoreMesh(core_axis_name="c", subcore_axis_name="s"),
        scratch_types=[
            pltpu.VMEM((TPW,), jnp.int32),
            pltpu.VMEM((TPW,), jnp.int32),
            [pltpu.VMEM((CT, D), jnp.float32)] * 2,
            [pltpu.VMEM((CT, D), jnp.float32)] * 2,
            [pltpu.VMEM((CT, D), jnp.float32)] * 2,
            pltpu.SemaphoreType.DMA,
            pltpu.SemaphoreType.DMA,
        ],
    )


def _combine(ygp, pos1, pos2):
    return _make_combine()(ygp, pos1, pos2)


@jax.jit
def kernel(x, gate_w, gate_b, expert_w, expert_b):
    itop, wtop, xp = _gate(x, gate_w, gate_b)
    row_token, row_weight, block_expert, pos1, pos2 = _route(itop, wtop)
    xgp = _gather_rows(xp, row_token)
    yg = _grouped_mm_half(xgp, expert_w, expert_b, row_weight,
                          block_expert, 0)
    return _combine(yg, pos1, pos2)


# speedup vs baseline: 1.0249x; 1.0249x over previous
import functools

import jax
import jax.numpy as jnp
from jax import lax
from jax.experimental import pallas as pl
from jax.experimental.pallas import tpu as pltpu
from jax.experimental.pallas import tpu_sc as plsc

N, D, E, TOP_K = 8192, 2048, 8, 2
H = D // 2
BLK = 256
M = TOP_K * N + E * BLK
NB = M // BLK
GB = 1024


def _gate_body(x_ref, gw_ref, gb_ref, itop_ref, wtop_ref, xp_ref):
    x = x_ref[...]
    logits = jax.lax.dot_general(
        x, gw_ref[...], (((1,), (1,)), ((), ())),
        preferred_element_type=jnp.float32) + gb_ref[...]
    probs = jax.nn.softmax(logits, axis=-1)
    cols = jax.lax.broadcasted_iota(jnp.int32, probs.shape, 1)
    i1 = jnp.argmax(probs, axis=-1, keepdims=True)
    p1 = jnp.max(probs, axis=-1, keepdims=True)
    pm = jnp.where(cols == i1, -jnp.inf, probs)
    i2 = jnp.argmax(pm, axis=-1, keepdims=True)
    p2 = jnp.max(pm, axis=-1, keepdims=True)
    itop_ref[...] = jnp.concatenate([i1, i2], axis=1)
    wtop_ref[...] = jnp.concatenate([p1, p2], axis=1)
    xbf = x.astype(jnp.bfloat16)
    lo = lax.bitcast_convert_type(xbf[:, :H], jnp.uint16).astype(jnp.uint32)
    hi = lax.bitcast_convert_type(xbf[:, H:], jnp.uint16).astype(jnp.uint32)
    xp_ref[...] = (hi << 16) | lo


def _gate(x, gate_w, gate_b):
    return pl.pallas_call(
        _gate_body,
        grid=(N // GB,),
        in_specs=[
            pl.BlockSpec((GB, D), lambda n: (n, 0)),
            pl.BlockSpec((E, D), lambda n: (0, 0)),
            pl.BlockSpec((1, E), lambda n: (0, 0)),
        ],
        out_specs=[
            pl.BlockSpec((GB, TOP_K), lambda n: (n, 0)),
            pl.BlockSpec((GB, TOP_K), lambda n: (n, 0)),
            pl.BlockSpec((GB, H), lambda n: (n, 0)),
        ],
        out_shape=[
            jax.ShapeDtypeStruct((N, TOP_K), jnp.int32),
            jax.ShapeDtypeStruct((N, TOP_K), jnp.float32),
            jax.ShapeDtypeStruct((N, H), jnp.uint32),
        ],
    )(x, gate_w, gate_b.reshape(1, E))


def _route(itop, wtop):
    flat_e = itop.reshape(-1)
    tok = jax.lax.iota(jnp.int32, TOP_K * N) // TOP_K
    oh = (flat_e[:, None] == jnp.arange(E, dtype=jnp.int32)[None, :])
    csum = jnp.cumsum(oh.astype(jnp.int32), axis=0)
    rank = jnp.take_along_axis(csum - 1, flat_e[:, None], axis=1)[:, 0]
    cnt = csum[-1]
    padded = ((cnt + BLK - 1) // BLK) * BLK
    off = jnp.concatenate([jnp.zeros((1,), jnp.int32),
                           jnp.cumsum(padded)[:-1].astype(jnp.int32)])
    pos = off[flat_e] + rank
    row_token = jnp.zeros((M,), jnp.int32).at[pos].set(tok)
    row_weight = jnp.zeros((M,), jnp.float32).at[pos].set(wtop.reshape(-1))
    block_expert = jnp.clip(
        jnp.searchsorted(off, jnp.arange(NB, dtype=jnp.int32) * BLK,
                         side="right") - 1, 0, E - 1).astype(jnp.int32)
    pos1 = pos[0::TOP_K]
    pos2 = pos[1::TOP_K]
    return row_token, row_weight, block_expert, pos1, pos2


NW = 32
CH = 32
G = 3


@functools.cache
def _make_gather(rows):
    rpw = rows // NW
    nchunk = rpw // CH

    def body(x_hbm, idx_hbm, out_hbm, idx_v, bufs, gsem, wsem):
        wid = lax.axis_index("s") * 2 + lax.axis_index("c")
        base = wid * rpw
        pltpu.sync_copy(idx_hbm.at[pl.ds(base, rpw)], idx_v)

        def group(g, _):
            cbase = g * G * CH
            cps = []
            for i in range(G):
                cb = cbase + i * CH
                cps.append(pltpu.async_copy(
                    x_hbm.at[idx_v.at[pl.ds(cb, CH)]], bufs[i], gsem))
            wps = []
            for i in range(G):
                cps[i].wait()
                wps.append(pltpu.async_copy(
                    bufs[i], out_hbm.at[pl.ds(base + cbase + i * CH, CH)],
                    wsem))
            for wp in wps:
                wp.wait()
            return 0

        lax.fori_loop(0, nchunk // G, group, 0)

    return pl.kernel(
        body,
        out_type=jax.ShapeDtypeStruct((rows, H), jnp.uint32),
        mesh=plsc.VectorSubcoreMesh(core_axis_name="c", subcore_axis_name="s"),
        scratch_types=[
            pltpu.VMEM((rpw,), jnp.int32),
            [pltpu.VMEM((CH, H), jnp.uint32)] * G,
            pltpu.SemaphoreType.DMA,
            pltpu.SemaphoreType.DMA,
        ],
    )


def _gather_rows(xp, row_token):
    return _make_gather(row_token.shape[0])(xp, row_token)


def _unpack_bf16(p):
    lo = lax.bitcast_convert_type(
        (p & jnp.uint32(0xFFFF)).astype(jnp.uint16), jnp.bfloat16)
    hi = lax.bitcast_convert_type(
        (p >> 16).astype(jnp.uint16), jnp.bfloat16)
    return lo, hi


def _mm_body(be_ref, xg_ref, w_ref, b_ref, rw_ref, yg_ref):
    xlo, xhi = _unpack_bf16(xg_ref[...])
    w = w_ref[0]
    wlo = w[:, :H].astype(jnp.bfloat16)
    whi = w[:, H:].astype(jnp.bfloat16)
    y = jax.lax.dot_general(xlo, wlo, (((1,), (1,)), ((), ())),
                            preferred_element_type=jnp.float32)
    y = y + jax.lax.dot_general(xhi, whi, (((1,), (1,)), ((), ())),
                                preferred_element_type=jnp.float32)
    yg_ref[...] = (y + b_ref[0]) * rw_ref[...]


def _mm_body_alias(be_ref, xg_ref, w_ref, b_ref, rw_ref, ygin_ref, yg_ref):
    _mm_body(be_ref, xg_ref, w_ref, b_ref, rw_ref, yg_ref)


def _grouped_mm_half(xgp_half, expert_w, expert_b, rw_half, be_half,
                     off, yg_in=None):
    nb = be_half.shape[0]
    in_specs = [
        pl.BlockSpec((BLK, H), lambda j, be: (j, 0)),
        pl.BlockSpec((1, D, D), lambda j, be: (be[j], 0, 0)),
        pl.BlockSpec((1, 1, D), lambda j, be: (be[j], 0, 0)),
        pl.BlockSpec((BLK, 1), lambda j, be: (j, 0)),
    ]
    args = [be_half, xgp_half, expert_w, expert_b.reshape(E, 1, D),
            rw_half.reshape(-1, 1)]
    aliases = {}
    if yg_in is not None:
        in_specs.append(pl.BlockSpec(memory_space=pl.ANY))
        args.append(yg_in)
        aliases = {5: 0}
    grid_spec = pltpu.PrefetchScalarGridSpec(
        num_scalar_prefetch=1,
        grid=(nb,),
        in_specs=in_specs,
        out_specs=pl.BlockSpec((BLK, D), lambda j, be: (off + j, 0)),
    )
    return pl.pallas_call(
        _mm_body if yg_in is None else _mm_body_alias,
        grid_spec=grid_spec,
        out_shape=jax.ShapeDtypeStruct((M, D), jnp.float32),
        input_output_aliases=aliases,
    )(*args)


TPW = N // NW
CT = 8
NTC = TPW // CT


def _combine_body(yg_hbm, p1_hbm, p2_hbm, out_hbm,
                  p1v, p2v, bufa, bufb, obuf, gsem, wsem):
    wid = lax.axis_index("s") * 2 + lax.axis_index("c")
    tbase = wid * TPW
    pltpu.sync_copy(p1_hbm.at[pl.ds(tbase, TPW)], p1v)
    pltpu.sync_copy(p2_hbm.at[pl.ds(tbase, TPW)], p2v)

    def compute(a_ref, b_ref, o_ref):
        def kloop(k, _):
            kk = k * 16
            for r in range(CT):
                o_ref[r, pl.ds(kk, 16)] = (
                    a_ref[r, pl.ds(kk, 16)] + b_ref[r, pl.ds(kk, 16)])
            return 0
        lax.fori_loop(0, D // 16, kloop, 0)

    def pair(j, _):
        gps = []
        for s in range(2):
            c = (j * 2 + s) * CT
            gps.append((
                pltpu.async_copy(yg_hbm.at[p1v.at[pl.ds(c, CT)]], bufa[s], gsem),
                pltpu.async_copy(yg_hbm.at[p2v.at[pl.ds(c, CT)]], bufb[s], gsem)))
        wps = []
        for s in range(2):
            c = (j * 2 + s) * CT
            gps[s][0].wait()
            gps[s][1].wait()
            compute(bufa[s], bufb[s], obuf[s])
            wps.append(pltpu.async_copy(
                obuf[s], out_hbm.at[pl.ds(tbase + c, CT)], wsem))
        for wp in wps:
            wp.wait()
        return 0

    lax.fori_loop(0, NTC // 2, pair, 0)


@functools.cache
def _make_combine():
    return pl.kernel(
        _combine_body,
        out_type=jax.ShapeDtypeStruct((N, D), jnp.float32),
        mesh=plsc.VectorSubcoreMesh(core_axis_name="c", subcore_axis_name="s"),
        scratch_types=[
            pltpu.VMEM((TPW,), jnp.int32),
            pltpu.VMEM((TPW,), jnp.int32),
            [pltpu.VMEM((CT, D), jnp.float32)] * 2,
            [pltpu.VMEM((CT, D), jnp.float32)] * 2,
            [pltpu.VMEM((CT, D), jnp.float32)] * 2,
            pltpu.SemaphoreType.DMA,
            pltpu.SemaphoreType.DMA,
        ],
    )


def _combine(ygp, pos1, pos2):
    return _make_combine()(ygp, pos1, pos2)


M2 = M // 2
NB2 = NB // 2


@jax.jit
def kernel(x, gate_w, gate_b, expert_w, expert_b):
    itop, wtop, xp = _gate(x, gate_w, gate_b)
    row_token, row_weight, block_expert, pos1, pos2 = _route(itop, wtop)
    xgp1 = _gather_rows(xp, row_token[:M2])
    xgp2 = _gather_rows(xp, row_token[M2:])
    yg1 = _grouped_mm_half(xgp1, expert_w, expert_b, row_weight[:M2],
                           block_expert[:NB2], 0)
    yg = _grouped_mm_half(xgp2, expert_w, expert_b, row_weight[M2:],
                          block_expert[NB2:], NB2, yg_in=yg1)
    return _combine(yg, pos1, pos2)
